# baseline (speedup 1.0000x reference)
import functools
import math

import jax
import jax.numpy as jnp
from jax import lax
from jax.experimental import pallas as pl
from jax.experimental.pallas import tpu as pltpu
from jax.experimental.pallas import tpu_sc as plsc

D_MODEL_ = 64
SCALE_ = math.sqrt(D_MODEL_)

_info = plsc.get_sparse_core_info()
_NC, _NS, _L = _info.num_cores, _info.num_subcores, _info.num_lanes
_NW = _NC * _NS

_CH = 128


def _make_kernel(B, V, D):
    assert B % (_NW * _CH) == 0
    b_per_w = B // _NW
    n_chunks = b_per_w // _CH
    mesh = plsc.VectorSubcoreMesh(core_axis_name="c", subcore_axis_name="s")

    @functools.partial(
        pl.kernel,
        mesh=mesh,
        out_type=jax.ShapeDtypeStruct((B, D), jnp.float32),
        scratch_types=[
            pltpu.VMEM((_CH,), jnp.int32),
            pltpu.VMEM((_CH, D), jnp.float32),
            pltpu.SemaphoreType.DMA,
        ],
        compiler_params=pltpu.CompilerParams(use_tc_tiling_on_sc=False),
    )
    def emb_kernel(idx_hbm, table_hbm, out_hbm, idx_v, rows_v, sem):
        wid = lax.axis_index("s") * _NC + lax.axis_index("c")
        base = wid * b_per_w

        def chunk_body(c, carry):
            start = base + c * _CH
            pltpu.sync_copy(idx_hbm.at[pl.ds(start, _CH)], idx_v)
            pltpu.async_copy(table_hbm.at[idx_v], rows_v, sem).wait()

            def row_body(i, carry2):
                for j in range(D // _L):
                    sl = pl.ds(j * _L, _L)
                    rows_v[i, sl] = rows_v[i, sl] * SCALE_
                return carry2

            lax.fori_loop(0, _CH, row_body, 0, unroll=2)
            pltpu.sync_copy(rows_v, out_hbm.at[pl.ds(start, _CH)])
            return carry

        lax.fori_loop(0, n_chunks, chunk_body, 0)

    return emb_kernel


def kernel(x, table):
    B = x.size
    V, D = table.shape
    out = _make_kernel(B, V, D)(x.reshape(B).astype(jnp.int32), table)
    return out.reshape(x.shape + (D,))
